# baseline (speedup 1.0000x reference)
import functools

import jax
import jax.numpy as jnp
from jax import lax
from jax.experimental import pallas as pl
from jax.experimental.pallas import tpu as pltpu
from jax.experimental.pallas import tpu_sc as plsc

N = 10000
E = 320000
NUM_GRAPHS = 64
EPS = 1e-7

NC = 2
NS = 16
D = 64
EPB = 80
E_PER_CORE = E // NC
E_PER_SUB = E // (NC * NS)
BLOCKS = E_PER_SUB // EPB
N_PAD = 10240
ROWS_PER_SUB = N_PAD // NS
ZROWS = 128


def _edge_body(h_hbm, e_hbm, src_hbm, dst_hbm, out_hbm,
               srcb, dstb, hrows, erows, cmb, zbuf, acc):
    c = lax.axis_index("c")
    s = lax.axis_index("s")

    zv = jnp.zeros((16,), jnp.float32)

    @pl.loop(0, ZROWS)
    def _(r):
        for g in range(2 * D // 16):
            zbuf[r, pl.ds(g * 16, 16)] = zv

    for k in range(ROWS_PER_SUB // ZROWS):
        pltpu.sync_copy(
            zbuf, acc.at[pl.ds(s * ROWS_PER_SUB + k * ZROWS, ZROWS)])
    plsc.subcore_barrier()

    base = c * E_PER_CORE + s * E_PER_SUB

    @pl.loop(0, BLOCKS)
    def _(j):
        off = base + j * EPB
        pltpu.sync_copy(src_hbm.at[pl.ds(off, EPB)], srcb)
        pltpu.sync_copy(dst_hbm.at[pl.ds(off, EPB)], dstb)
        pltpu.sync_copy(h_hbm.at[srcb], hrows)
        pltpu.sync_copy(e_hbm.at[pl.ds(off, EPB)], erows)

        @pl.loop(0, EPB)
        def _(r):
            for g in range(D // 16):
                sl = pl.ds(g * 16, 16)
                m = jnp.maximum(hrows[r, sl] + erows[r, sl], 0.0) + EPS
                w = jnp.exp(m)
                cmb[r, sl] = w
                cmb[r, pl.ds(D + g * 16, 16)] = m * w

        pltpu.sync_copy(cmb, acc.at[dstb], add=True)

    plsc.subcore_barrier()
    pltpu.sync_copy(
        acc.at[pl.ds(s * ROWS_PER_SUB, ROWS_PER_SUB)],
        out_hbm.at[c, pl.ds(s * ROWS_PER_SUB, ROWS_PER_SUB)])


@jax.jit
def _edge_pass(h, e, src, dst):
    mesh = plsc.VectorSubcoreMesh(core_axis_name="c", subcore_axis_name="s")
    f = pl.kernel(
        _edge_body,
        out_type=jax.ShapeDtypeStruct((NC, N_PAD, 2 * D), jnp.float32),
        mesh=mesh,
        scratch_types=[
            pltpu.VMEM((EPB,), jnp.int32),
            pltpu.VMEM((EPB,), jnp.int32),
            pltpu.VMEM((EPB, D), jnp.float32),
            pltpu.VMEM((EPB, D), jnp.float32),
            pltpu.VMEM((EPB, 2 * D), jnp.float32),
            pltpu.VMEM((ZROWS, 2 * D), jnp.float32),
            pltpu.VMEM_SHARED((N_PAD, 2 * D), jnp.float32),
        ],
        compiler_params=pltpu.CompilerParams(use_tc_tiling_on_sc=False),
    )
    return f(h, e, src, dst)


def _aggregate(h, e, src, dst):
    p = _edge_pass(h, e, src, dst)
    p = p[0] + p[1]
    return p[:N, D:] / (p[:N, :D] + 1e-16)


def _bn(h, g, bt):
    mu = h.mean(axis=0)
    var = h.var(axis=0)
    return (h - mu) / jnp.sqrt(var + 1e-5) * g + bt


def _conv(p, x, src, dst, edge_attr):
    if 'Wsrc' in p:
        h = x @ p['Wsrc'] + p['bsrc']
        xd = x @ p['Wdst'] + p['bdst']
    else:
        h = x
        xd = x
    e = edge_attr @ p['We'] + p['be']
    d = h.shape[1]
    if d == D:
        agg = _aggregate(h, e, src, dst)
    else:
        parts = [
            _aggregate(h[:, k:k + D], e[:, k:k + D], src, dst)
            for k in range(0, d, D)
        ]
        agg = jnp.concatenate(parts, axis=1)
    out = agg + xd
    h2 = out @ p['W1'] + p['b1']
    h2 = jax.nn.relu(_bn(h2, p['g'], p['bt']))
    return h2 @ p['W2'] + p['b2']


def kernel(x, edge_index, edge_attr, batch, params):
    src, dst = edge_index[0], edge_index[1]
    h = jax.nn.relu(_conv(params['conv1'], x, src, dst, edge_attr))
    h = jax.nn.relu(_conv(params['conv2'], h, src, dst, edge_attr))
    h = jax.nn.relu(_conv(params['conv3'], h, src, dst, edge_attr))
    onehot = (batch[:, None] == jnp.arange(NUM_GRAPHS)[None, :]).astype(jnp.float32)
    s = lax.dot_general(onehot, h, (((0,), (0,)), ((), ())))
    cnt = jnp.sum(onehot, axis=0)[:, None]
    pooled = s / jnp.maximum(cnt, 1.0)
    h = pooled @ params['d1W'] + params['d1b']
    h = h @ params['d2W'] + params['d2b']
    return jax.nn.log_softmax(h, axis=-1)


# speedup vs baseline: 1.4786x; 1.4786x over previous
import functools

import jax
import jax.numpy as jnp
from jax import lax
from jax.experimental import pallas as pl
from jax.experimental.pallas import tpu as pltpu
from jax.experimental.pallas import tpu_sc as plsc

N = 10000
E = 320000
NUM_GRAPHS = 64
EPS = 1e-7

NC = 2
NS = 16
D = 64
EPB = 80
E_PER_CORE = E // NC
E_PER_SUB = E // (NC * NS)
BLOCKS = E_PER_SUB // EPB
N_PAD = 10112
ROWS_PER_SUB = N_PAD // NS
ZROWS = 128


def _edge_body(h_hbm, e_hbm, src_hbm, dst_hbm, out_hbm,
               srcb, dstb, dstsc, hrows, erows, wbuf, mwbuf, zbuf, acc,
               s_src, s_dst, s_h, s_e, s_sw, s_sm):
    c = lax.axis_index("c")
    s = lax.axis_index("s")
    base = c * E_PER_CORE + s * E_PER_SUB

    def idx_start(j, b):
        off = base + j * EPB
        pltpu.async_copy(src_hbm.at[pl.ds(off, EPB)], srcb.at[b], s_src.at[b])
        pltpu.async_copy(dst_hbm.at[pl.ds(off, EPB)], dstb.at[b], s_dst.at[b])

    def idx_wait(b):
        pltpu.make_async_copy(src_hbm.at[pl.ds(0, EPB)], srcb.at[b],
                              s_src.at[b]).wait()
        pltpu.make_async_copy(dst_hbm.at[pl.ds(0, EPB)], dstb.at[b],
                              s_dst.at[b]).wait()

    def gather_start(j, b):
        off = base + j * EPB
        pltpu.async_copy(h_hbm.at[srcb.at[b]], hrows.at[b], s_h.at[b])
        pltpu.async_copy(e_hbm.at[pl.ds(off, EPB)], erows.at[b], s_e.at[b])

    def gather_wait(b):
        pltpu.make_async_copy(h_hbm.at[srcb.at[b]], hrows.at[b],
                              s_h.at[b]).wait()
        pltpu.make_async_copy(e_hbm.at[pl.ds(0, EPB)], erows.at[b],
                              s_e.at[b]).wait()

    def scat_start(b):
        pltpu.async_copy(wbuf.at[b], acc.at[0].at[dstsc.at[b]], s_sw.at[b],
                         add=True)
        pltpu.async_copy(mwbuf.at[b], acc.at[1].at[dstsc.at[b]], s_sm.at[b],
                         add=True)

    def scat_wait(b):
        pltpu.make_async_copy(wbuf.at[b], acc.at[0].at[dstsc.at[b]],
                              s_sw.at[b]).wait()
        pltpu.make_async_copy(mwbuf.at[b], acc.at[1].at[dstsc.at[b]],
                              s_sm.at[b]).wait()

    idx_start(0, 0)
    idx_start(1, 1)

    zv = jnp.zeros((16,), jnp.float32)

    @pl.loop(0, ZROWS)
    def _(r):
        for g in range(D // 16):
            zbuf[r, pl.ds(g * 16, 16)] = zv

    for a in range(2):
        for k in range(ROWS_PER_SUB // ZROWS):
            pltpu.sync_copy(
                zbuf, acc.at[a, pl.ds(s * ROWS_PER_SUB + k * ZROWS, ZROWS)])
        rem = ROWS_PER_SUB % ZROWS
        if rem:
            pltpu.sync_copy(
                zbuf.at[pl.ds(0, rem)],
                acc.at[a, pl.ds(s * ROWS_PER_SUB
                                + (ROWS_PER_SUB // ZROWS) * ZROWS, rem)])
    plsc.subcore_barrier()

    idx_wait(0)
    gather_start(0, 0)

    @pl.loop(0, BLOCKS)
    def _(j):
        b = jax.lax.rem(j, 2)
        gather_wait(b)

        @pl.when(j >= 2)
        def _():
            scat_wait(b)

        for g in range(EPB // 16):
            sl = pl.ds(g * 16, 16)
            dstsc[b, sl] = dstb[b, sl]

        @pl.when(j + 2 < BLOCKS)
        def _():
            idx_start(j + 2, b)

        @pl.when(j + 1 < BLOCKS)
        def _():
            idx_wait(1 - b)
            gather_start(j + 1, 1 - b)

        @pl.loop(0, EPB)
        def _(r):
            for g in range(D // 16):
                sl = pl.ds(g * 16, 16)
                m = jnp.maximum(hrows[b, r, sl] + erows[b, r, sl], 0.0) + EPS
                w = jnp.exp(m)
                wbuf[b, r, sl] = w
                mwbuf[b, r, sl] = m * w

        scat_start(b)

    scat_wait((BLOCKS - 2) % 2)
    scat_wait((BLOCKS - 1) % 2)

    plsc.subcore_barrier()
    for a in range(2):
        pltpu.sync_copy(
            acc.at[a, pl.ds(s * ROWS_PER_SUB, ROWS_PER_SUB)],
            out_hbm.at[c, a, pl.ds(s * ROWS_PER_SUB, ROWS_PER_SUB)])


@jax.jit
def _edge_pass(h, e, src, dst):
    mesh = plsc.VectorSubcoreMesh(core_axis_name="c", subcore_axis_name="s")
    f = pl.kernel(
        _edge_body,
        out_type=jax.ShapeDtypeStruct((NC, 2, N_PAD, D), jnp.float32),
        mesh=mesh,
        scratch_types=[
            pltpu.VMEM((2, EPB), jnp.int32),
            pltpu.VMEM((2, EPB), jnp.int32),
            pltpu.VMEM((2, EPB), jnp.int32),
            pltpu.VMEM((2, EPB, D), jnp.float32),
            pltpu.VMEM((2, EPB, D), jnp.float32),
            pltpu.VMEM((2, EPB, D), jnp.float32),
            pltpu.VMEM((2, EPB, D), jnp.float32),
            pltpu.VMEM((ZROWS, D), jnp.float32),
            pltpu.VMEM_SHARED((2, N_PAD, D), jnp.float32),
            pltpu.SemaphoreType.DMA((2,)),
            pltpu.SemaphoreType.DMA((2,)),
            pltpu.SemaphoreType.DMA((2,)),
            pltpu.SemaphoreType.DMA((2,)),
            pltpu.SemaphoreType.DMA((2,)),
            pltpu.SemaphoreType.DMA((2,)),
        ],
        compiler_params=pltpu.CompilerParams(use_tc_tiling_on_sc=False),
    )
    return f(h, e, src, dst)


def _aggregate(h, e, src, dst):
    p = _edge_pass(h, e, src, dst)
    p = p[0] + p[1]
    return p[1, :N] / (p[0, :N] + 1e-16)


def _bn(h, g, bt):
    mu = h.mean(axis=0)
    var = h.var(axis=0)
    return (h - mu) / jnp.sqrt(var + 1e-5) * g + bt


def _conv(p, x, src, dst, edge_attr):
    if 'Wsrc' in p:
        h = x @ p['Wsrc'] + p['bsrc']
        xd = x @ p['Wdst'] + p['bdst']
    else:
        h = x
        xd = x
    e = edge_attr @ p['We'] + p['be']
    d = h.shape[1]
    if d == D:
        agg = _aggregate(h, e, src, dst)
    else:
        parts = [
            _aggregate(h[:, k:k + D], e[:, k:k + D], src, dst)
            for k in range(0, d, D)
        ]
        agg = jnp.concatenate(parts, axis=1)
    out = agg + xd
    h2 = out @ p['W1'] + p['b1']
    h2 = jax.nn.relu(_bn(h2, p['g'], p['bt']))
    return h2 @ p['W2'] + p['b2']


def kernel(x, edge_index, edge_attr, batch, params):
    src, dst = edge_index[0], edge_index[1]
    h = jax.nn.relu(_conv(params['conv1'], x, src, dst, edge_attr))
    h = jax.nn.relu(_conv(params['conv2'], h, src, dst, edge_attr))
    h = jax.nn.relu(_conv(params['conv3'], h, src, dst, edge_attr))
    onehot = (batch[:, None] == jnp.arange(NUM_GRAPHS)[None, :]).astype(jnp.float32)
    s = lax.dot_general(onehot, h, (((0,), (0,)), ((), ())))
    cnt = jnp.sum(onehot, axis=0)[:, None]
    pooled = s / jnp.maximum(cnt, 1.0)
    h = pooled @ params['d1W'] + params['d1b']
    h = h @ params['d2W'] + params['d2b']
    return jax.nn.log_softmax(h, axis=-1)


# speedup vs baseline: 1.7371x; 1.1748x over previous
import functools

import jax
import jax.numpy as jnp
from jax import lax
from jax.experimental import pallas as pl
from jax.experimental.pallas import tpu as pltpu
from jax.experimental.pallas import tpu_sc as plsc

N = 10000
E = 320000
NUM_GRAPHS = 64
EPS = 1e-7

NC = 2
NS = 16
D = 64
EPB = 128
E_PER_SUB = 10240
E_PAD = NC * NS * E_PER_SUB
E_PER_CORE = E_PAD // NC
BLOCKS = E_PER_SUB // EPB
N_PAD = 10112
ROWS_PER_SUB = N_PAD // NS
ZROWS = 128
JUNK_ROW = N_PAD - 1


def _edge_body(h_hbm, e_hbm, src_hbm, dst_hbm, out_hbm,
               srcb, dstb, hrows, erows, wbuf, mwbuf, zbuf,
               acc, s_src, s_dst, s_h, s_e, s_sw, s_sm):
    c = lax.axis_index("c")
    s = lax.axis_index("s")
    base = c * E_PER_CORE + s * E_PER_SUB

    zv = jnp.zeros((16,), jnp.float32)

    @pl.loop(0, ZROWS)
    def _(r):
        for g in range(D // 16):
            zbuf[r, pl.ds(g * 16, 16)] = zv

    for a in range(2):
        for k in range(ROWS_PER_SUB // ZROWS):
            pltpu.sync_copy(
                zbuf, acc.at[a, pl.ds(s * ROWS_PER_SUB + k * ZROWS, ZROWS)])
        rem = ROWS_PER_SUB % ZROWS
        if rem:
            pltpu.sync_copy(
                zbuf.at[pl.ds(0, rem)],
                acc.at[a, pl.ds(s * ROWS_PER_SUB
                                + (ROWS_PER_SUB // ZROWS) * ZROWS, rem)])
    plsc.subcore_barrier()

    @pl.loop(0, BLOCKS)
    def _(j):
        off = base + j * EPB
        cp_s = pltpu.async_copy(src_hbm.at[pl.ds(off, EPB)], srcb, s_src)
        cp_d = pltpu.async_copy(dst_hbm.at[pl.ds(off, EPB)], dstb, s_dst)
        cp_s.wait()
        cp_d.wait()
        cp_h = pltpu.async_copy(h_hbm.at[srcb], hrows, s_h)
        cp_e = pltpu.async_copy(e_hbm.at[pl.ds(off, EPB)], erows, s_e)
        cp_h.wait()
        cp_e.wait()

        @pl.loop(0, EPB)
        def _(r):
            for g in range(D // 16):
                sl = pl.ds(g * 16, 16)
                m = jnp.maximum(hrows[r, sl] + erows[r, sl], 0.0) + EPS
                w = jnp.exp(m)
                wbuf[r, sl] = w
                mwbuf[r, sl] = m * w

        cp_w = pltpu.async_copy(wbuf, acc.at[0].at[dstb], s_sw, add=True)
        cp_m = pltpu.async_copy(mwbuf, acc.at[1].at[dstb], s_sm, add=True)
        cp_w.wait()
        cp_m.wait()

    plsc.subcore_barrier()
    for a in range(2):
        pltpu.sync_copy(
            acc.at[a, pl.ds(s * ROWS_PER_SUB, ROWS_PER_SUB)],
            out_hbm.at[c, a, pl.ds(s * ROWS_PER_SUB, ROWS_PER_SUB)])


@jax.jit
def _edge_pass(h, e, src, dst):
    mesh = plsc.VectorSubcoreMesh(core_axis_name="c", subcore_axis_name="s")
    f = pl.kernel(
        _edge_body,
        out_type=jax.ShapeDtypeStruct((NC, 2, N_PAD, D), jnp.float32),
        mesh=mesh,
        scratch_types=[
            pltpu.VMEM((EPB,), jnp.int32),
            pltpu.VMEM((EPB,), jnp.int32),
            pltpu.VMEM((EPB, D), jnp.float32),
            pltpu.VMEM((EPB, D), jnp.float32),
            pltpu.VMEM((EPB, D), jnp.float32),
            pltpu.VMEM((EPB, D), jnp.float32),
            pltpu.VMEM((ZROWS, D), jnp.float32),
            pltpu.VMEM_SHARED((2, N_PAD, D), jnp.float32),
            pltpu.SemaphoreType.DMA,
            pltpu.SemaphoreType.DMA,
            pltpu.SemaphoreType.DMA,
            pltpu.SemaphoreType.DMA,
            pltpu.SemaphoreType.DMA,
            pltpu.SemaphoreType.DMA,
        ],
        compiler_params=pltpu.CompilerParams(use_tc_tiling_on_sc=False),
    )
    return f(h, e, src, dst)


def _aggregate(h, e, src, dst):
    p = _edge_pass(h, e, src, dst)
    p = p[0] + p[1]
    return p[1, :N] / (p[0, :N] + 1e-16)


def _bn(h, g, bt):
    mu = h.mean(axis=0)
    var = h.var(axis=0)
    return (h - mu) / jnp.sqrt(var + 1e-5) * g + bt


def _conv(p, x, src, dst, edge_attr):
    if 'Wsrc' in p:
        h = x @ p['Wsrc'] + p['bsrc']
        xd = x @ p['Wdst'] + p['bdst']
    else:
        h = x
        xd = x
    e = edge_attr @ p['We'] + p['be']
    d = h.shape[1]
    if d == D:
        agg = _aggregate(h, e, src, dst)
    else:
        parts = [
            _aggregate(h[:, k:k + D], e[:, k:k + D], src, dst)
            for k in range(0, d, D)
        ]
        agg = jnp.concatenate(parts, axis=1)
    out = agg + xd
    h2 = out @ p['W1'] + p['b1']
    h2 = jax.nn.relu(_bn(h2, p['g'], p['bt']))
    return h2 @ p['W2'] + p['b2']


def kernel(x, edge_index, edge_attr, batch, params):
    src, dst = edge_index[0], edge_index[1]
    npad = E_PAD - E
    src = jnp.concatenate([src, jnp.zeros((npad,), jnp.int32)])
    dst = jnp.concatenate([dst, jnp.full((npad,), JUNK_ROW, jnp.int32)])
    edge_attr = jnp.pad(edge_attr, ((0, npad), (0, 0)))
    h = jax.nn.relu(_conv(params['conv1'], x, src, dst, edge_attr))
    h = jax.nn.relu(_conv(params['conv2'], h, src, dst, edge_attr))
    h = jax.nn.relu(_conv(params['conv3'], h, src, dst, edge_attr))
    onehot = (batch[:, None] == jnp.arange(NUM_GRAPHS)[None, :]).astype(jnp.float32)
    s = lax.dot_general(onehot, h, (((0,), (0,)), ((), ())))
    cnt = jnp.sum(onehot, axis=0)[:, None]
    pooled = s / jnp.maximum(cnt, 1.0)
    h = pooled @ params['d1W'] + params['d1b']
    h = h @ params['d2W'] + params['d2b']
    return jax.nn.log_softmax(h, axis=-1)
